# baseline (speedup 1.0000x reference)
import functools

import jax
import jax.numpy as jnp
from jax import lax
from jax.experimental import pallas as pl
from jax.experimental.pallas import tpu as pltpu
from jax.experimental.pallas import tpu_sc as plsc

N = 50000
E = 800000
G = 1000
HID = 64
EMB = 32

NC = 2
NS = 16

_MESH = dict(core_axis_name="c", subcore_axis_name="s", num_cores=NC,
             num_subcores=NS)
_SC_PARAMS = dict(
    mesh=plsc.VectorSubcoreMesh(**_MESH),
    compiler_params=pltpu.CompilerParams(use_tc_tiling_on_sc=False),
)


def _dot_t(a, w):
    return lax.dot_general(a, w, (((1,), (1,)), ((), ())),
                           preferred_element_type=jnp.float32)


def _fill_vmem_rows(ref, nrows, ncols, val):
    v16 = jnp.full((16,), val, jnp.float32)

    def body(i, _):
        for j in range(ncols // 16):
            ref[i, pl.ds(j * 16, 16)] = v16
        return 0

    lax.fori_loop(0, nrows, body, 0, unroll=False)


_CP = 3128
_CPLAST = N - (NS - 1) * _CP


def _per_tile_rows(s, fn):
    @pl.when(s < NS - 1)
    def _():
        fn(s * _CP, _CP)

    @pl.when(s == NS - 1)
    def _():
        fn((NS - 1) * _CP, _CPLAST)


def _copy_zero_slice(zbuf, acc, base, total, chunk):
    nfull = total // chunk
    rem = total - nfull * chunk
    for k in range(nfull):
        pltpu.sync_copy(zbuf, acc.at[pl.ds(base + k * chunk, chunk)])
    if rem:
        pltpu.sync_copy(zbuf.at[pl.ds(0, rem)],
                        acc.at[pl.ds(base + nfull * chunk, rem)])


def _sc_agg16(h0p, src, dst):
    C = 800
    NCHUNK = E // C
    NW = NC * NS
    JMAX = (NCHUNK + NW - 1) // NW

    @functools.partial(
        pl.kernel,
        out_type=jax.ShapeDtypeStruct((N, 4, 16), jnp.float32),
        scratch_types=[
            pltpu.VMEM((C,), jnp.int32),
            pltpu.VMEM((C,), jnp.int32),
            pltpu.VMEM((C, 1, 16), jnp.float32),
            pltpu.VMEM_SHARED((N, 1, 16), jnp.float32),
            pltpu.VMEM_SHARED((N, 1, 16), jnp.float32),
        ],
        **_SC_PARAMS,
    )
    def k(h_hbm, src_hbm, dst_hbm, out_hbm, sidx, didx, rows, tab, acc):
        c = lax.axis_index("c")
        s = lax.axis_index("s")
        w = s * NC + c

        def zrow(i, _):
            rows[i, 0, :] = jnp.zeros((16,), jnp.float32)
            return 0

        lax.fori_loop(0, C, zrow, 0, unroll=False)
        _per_tile_rows(s, lambda base, n: (
            pltpu.sync_copy(h_hbm.at[pl.ds(base, n)], tab.at[pl.ds(base, n)]),
            _copy_zero_slice(rows, acc, base, n, C)))
        plsc.subcore_barrier()

        def body(j, _):
            ch = w + NW * j

            @pl.when(ch < NCHUNK)
            def _():
                off = ch * C
                pltpu.sync_copy(src_hbm.at[pl.ds(off, C)], sidx)
                pltpu.sync_copy(dst_hbm.at[pl.ds(off, C)], didx)
                pltpu.sync_copy(tab.at[sidx], rows)
                pltpu.sync_copy(rows, acc.at[didx], add=True)

            return 0

        lax.fori_loop(0, JMAX, body, 0, unroll=False)
        plsc.subcore_barrier()
        _per_tile_rows(s, lambda base, n: pltpu.sync_copy(
            acc.at[pl.ds(base, n)], out_hbm.at[pl.ds(base, n),
                                               pl.ds(c, 1)]))

    return k(h0p, src, dst)


def _sc_agg_split(h4, src, dst):
    C = 800
    NCHUNK = E // C
    JMAX = (NCHUNK + NS - 1) // NS

    @functools.partial(
        pl.kernel,
        out_type=jax.ShapeDtypeStruct((N, 4, 16), jnp.float32),
        scratch_types=[
            pltpu.VMEM((C,), jnp.int32),
            pltpu.VMEM((C,), jnp.int32),
            pltpu.VMEM((C, 1, 16), jnp.float32),
            pltpu.VMEM_SHARED((N, 1, 16), jnp.float32),
            pltpu.VMEM_SHARED((N, 1, 16), jnp.float32),
        ],
        **_SC_PARAMS,
    )
    def k(h_hbm, src_hbm, dst_hbm, out_hbm, sidx, didx, rows, tab, acc):
        c = lax.axis_index("c")
        s = lax.axis_index("s")

        def zrow(i, _):
            rows[i, 0, :] = jnp.zeros((16,), jnp.float32)
            return 0

        for p in range(2):
            pv = c * 2 + p
            lax.fori_loop(0, C, zrow, 0, unroll=False)
            _per_tile_rows(s, lambda base, n, pv=pv: (
                pltpu.sync_copy(h_hbm.at[pl.ds(base, n), pl.ds(pv, 1)],
                                tab.at[pl.ds(base, n)]),
                _copy_zero_slice(rows, acc, base, n, C)))
            plsc.subcore_barrier()

            def body(j, _):
                ch = s + NS * j

                @pl.when(ch < NCHUNK)
                def _():
                    off = ch * C
                    pltpu.sync_copy(src_hbm.at[pl.ds(off, C)], sidx)
                    pltpu.sync_copy(dst_hbm.at[pl.ds(off, C)], didx)
                    pltpu.sync_copy(tab.at[sidx], rows)
                    pltpu.sync_copy(rows, acc.at[didx], add=True)

                return 0

            lax.fori_loop(0, JMAX, body, 0, unroll=False)
            plsc.subcore_barrier()
            _per_tile_rows(s, lambda base, n, pv=pv: pltpu.sync_copy(
                acc.at[pl.ds(base, n)], out_hbm.at[pl.ds(base, n),
                                                   pl.ds(pv, 1)]))
            if p == 0:
                plsc.subcore_barrier()

    return k(h4, src, dst)


def _sc_pool(h3, batch):
    CH = 400
    NCHUNK = N // CH
    NW = NC * NS
    JMAX = (NCHUNK + NW - 1) // NW

    @functools.partial(
        pl.kernel,
        out_type=(jax.ShapeDtypeStruct((NC, G, HID), jnp.float32),
                  jax.ShapeDtypeStruct((NC, G, 16), jnp.float32)),
        scratch_types=[
            pltpu.VMEM((CH,), jnp.int32),
            pltpu.VMEM((CH, HID), jnp.float32),
            pltpu.VMEM((CH, 16), jnp.float32),
            pltpu.VMEM((CH, 16), jnp.float32),
            pltpu.VMEM_SHARED((G, HID), jnp.float32),
            pltpu.VMEM_SHARED((G, 16), jnp.float32),
        ],
        **_SC_PARAMS,
    )
    def k(h_hbm, batch_hbm, outp_hbm, outc_hbm,
          bidx, rows, ones_v, zb16, accp, accc):
        c = lax.axis_index("c")
        s = lax.axis_index("s")
        w = s * NC + c
        _fill_vmem_rows(ones_v, CH, 16, 1.0)
        _fill_vmem_rows(rows, CH, HID, 0.0)
        _fill_vmem_rows(zb16, CH, 16, 0.0)

        @pl.when(s == 0)
        def _():
            _copy_zero_slice(rows, accp, 0, G, CH)

        @pl.when(s == 1)
        def _():
            _copy_zero_slice(zb16, accc, 0, G, CH)

        plsc.subcore_barrier()

        def body(j, _):
            ch = w + NW * j

            @pl.when(ch < NCHUNK)
            def _():
                off = ch * CH
                pltpu.sync_copy(batch_hbm.at[pl.ds(off, CH)], bidx)
                pltpu.sync_copy(h_hbm.at[pl.ds(off, CH)], rows)
                pltpu.sync_copy(rows, accp.at[bidx], add=True)
                pltpu.sync_copy(ones_v, accc.at[bidx], add=True)

            return 0

        lax.fori_loop(0, JMAX, body, 0, unroll=False)
        plsc.subcore_barrier()

        @pl.when(s == 0)
        def _():
            pltpu.sync_copy(accp, outp_hbm.at[c])

        @pl.when(s == 1)
        def _():
            pltpu.sync_copy(accc, outc_hbm.at[c])

    return k(h3, batch)


_BLK = 1000
_NB = N // _BLK


def _accum_stats(first, z, st_ref):
    st = jnp.concatenate([jnp.sum(z, axis=0)[None, :],
                          jnp.sum(z * z, axis=0)[None, :]], axis=0)

    @pl.when(first)
    def _():
        st_ref[...] = st

    @pl.when(jnp.logical_not(first))
    def _():
        st_ref[...] = st_ref[...] + st


def _tc_input_bn(x, g, b):
    def body(x_ref, g_ref, b_ref, o_ref, st):
        ph = pl.program_id(0)
        i = pl.program_id(1)
        xv = x_ref[...]

        @pl.when(ph == 0)
        def _():
            _accum_stats(i == 0, xv, st)

        @pl.when(ph == 1)
        def _():
            m = st[0] / N
            v = st[1] / N - m * m
            h = (xv - m) * lax.rsqrt(v + 1e-5) * g_ref[...] + b_ref[...]
            o_ref[...] = jnp.concatenate(
                [h, jnp.ones((_BLK, 1), jnp.float32),
                 jnp.zeros((_BLK, 11), jnp.float32)], axis=1)

    return pl.pallas_call(
        body, grid=(2, _NB),
        in_specs=[
            pl.BlockSpec((_BLK, 4), lambda ph, i: (i, 0)),
            pl.BlockSpec((4,), lambda ph, i: (0,)),
            pl.BlockSpec((4,), lambda ph, i: (0,)),
        ],
        out_specs=pl.BlockSpec((_BLK, 16), lambda ph, i: (i, 0)),
        out_shape=jax.ShapeDtypeStruct((N, 16), jnp.float32),
        scratch_shapes=[pltpu.VMEM((2, 4), jnp.float32)])(x, g, b)


def _tc_layer0(part0w, h0p, Wn, Ws, bc, g, b):
    def body(p_ref, h_ref, wn_ref, ws_ref, bc_ref, g_ref, b_ref,
             dg_ref, o_ref, st):
        ph = pl.program_id(0)
        i = pl.program_id(1)
        pv = p_ref[...]
        psum = pv[:, 0:16] + pv[:, 16:32]
        deginv = 1.0 / jnp.maximum(psum[:, 4], 1.0)
        dg_ref[...] = deginv[None, None, :]
        agg = psum[:, :4] * deginv[:, None]
        h0 = h_ref[...][:, :4]
        z = _dot_t(agg, wn_ref[...]) + _dot_t(h0, ws_ref[...]) + bc_ref[...]

        @pl.when(ph == 0)
        def _():
            _accum_stats(i == 0, z, st)

        @pl.when(ph == 1)
        def _():
            m = st[0] / N
            v = st[1] / N - m * m
            y = (z - m) * lax.rsqrt(v + 1e-5) * g_ref[...] + b_ref[...]
            o_ref[...] = jnp.maximum(y, 0.0)

    return pl.pallas_call(
        body, grid=(2, _NB),
        in_specs=[
            pl.BlockSpec((_BLK, HID), lambda ph, i: (i, 0)),
            pl.BlockSpec((_BLK, 16), lambda ph, i: (i, 0)),
            pl.BlockSpec((HID, 4), lambda ph, i: (0, 0)),
            pl.BlockSpec((HID, 4), lambda ph, i: (0, 0)),
            pl.BlockSpec((HID,), lambda ph, i: (0,)),
            pl.BlockSpec((HID,), lambda ph, i: (0,)),
            pl.BlockSpec((HID,), lambda ph, i: (0,)),
        ],
        out_specs=[
            pl.BlockSpec((1, 1, _BLK), lambda ph, i: (i, 0, 0)),
            pl.BlockSpec((_BLK, HID), lambda ph, i: (i, 0)),
        ],
        out_shape=[
            jax.ShapeDtypeStruct((_NB, 1, _BLK), jnp.float32),
            jax.ShapeDtypeStruct((N, HID), jnp.float32),
        ],
        scratch_shapes=[pltpu.VMEM((2, HID), jnp.float32)],
    )(part0w, h0p, Wn, Ws, bc, g, b)


def _tc_layer(sums, deginv, h, Wn, Ws, bc, g, b):
    def body(su_ref, dg_ref, h_ref, wn_ref, ws_ref, bc_ref, g_ref, b_ref,
             o_ref, st):
        ph = pl.program_id(0)
        i = pl.program_id(1)
        agg = su_ref[...] * dg_ref[0, 0][:, None]
        z = (_dot_t(agg, wn_ref[...]) + _dot_t(h_ref[...], ws_ref[...])
             + bc_ref[...])

        @pl.when(ph == 0)
        def _():
            _accum_stats(i == 0, z, st)

        @pl.when(ph == 1)
        def _():
            m = st[0] / N
            v = st[1] / N - m * m
            y = (z - m) * lax.rsqrt(v + 1e-5) * g_ref[...] + b_ref[...]
            o_ref[...] = jnp.maximum(y, 0.0)

    return pl.pallas_call(
        body, grid=(2, _NB),
        in_specs=[
            pl.BlockSpec((_BLK, HID), lambda ph, i: (i, 0)),
            pl.BlockSpec((1, 1, _BLK), lambda ph, i: (i, 0, 0)),
            pl.BlockSpec((_BLK, HID), lambda ph, i: (i, 0)),
            pl.BlockSpec((HID, HID), lambda ph, i: (0, 0)),
            pl.BlockSpec((HID, HID), lambda ph, i: (0, 0)),
            pl.BlockSpec((HID,), lambda ph, i: (0,)),
            pl.BlockSpec((HID,), lambda ph, i: (0,)),
            pl.BlockSpec((HID,), lambda ph, i: (0,)),
        ],
        out_specs=pl.BlockSpec((_BLK, HID), lambda ph, i: (i, 0)),
        out_shape=jax.ShapeDtypeStruct((N, HID), jnp.float32),
        scratch_shapes=[pltpu.VMEM((2, HID), jnp.float32)],
    )(sums, deginv, h, Wn, Ws, bc, g, b)


def _tc_readout(poolp, poolc, Wh0, bh0, Wh1, bh1):
    def body(p_ref, c_ref, w0_ref, b0_ref, w1_ref, b1_ref, o_ref):
        pooled = p_ref[0] + p_ref[1]
        cnt = jnp.maximum(c_ref[0][:, 0] + c_ref[1][:, 0], 1.0)
        mean = pooled / cnt[:, None]
        z = jnp.maximum(_dot_t(mean, w0_ref[...]) + b0_ref[...], 0.0)
        z2 = _dot_t(z, w1_ref[...]) + b1_ref[...]
        n = jnp.sqrt(jnp.sum(z2 * z2, axis=1, keepdims=True))
        o_ref[...] = z2 / jnp.maximum(n, 1e-12)

    return pl.pallas_call(
        body,
        out_shape=jax.ShapeDtypeStruct((G, EMB), jnp.float32))(
            poolp, poolc, Wh0, bh0, Wh1, bh1)


def kernel(x, edge_index, batch, bn_in_g, bn_in_b,
           Wn0, Ws0, bc0, bn_g0, bn_b0,
           Wn1, Ws1, bc1, bn_g1, bn_b1,
           Wn2, Ws2, bc2, bn_g2, bn_b2,
           Wh0, bh0, Wh1, bh1):
    src = edge_index[0]
    dst = edge_index[1]

    h0p = _tc_input_bn(x, bn_in_g, bn_in_b)

    part0 = _sc_agg16(h0p.reshape(N, 1, 16), src, dst)
    deginv, h1 = _tc_layer0(part0.reshape(N, HID), h0p, Wn0, Ws0, bc0,
                            bn_g0, bn_b0)

    sum1 = _sc_agg_split(h1.reshape(N, 4, 16), src, dst)
    h2 = _tc_layer(sum1.reshape(N, HID), deginv, h1, Wn1, Ws1, bc1,
                   bn_g1, bn_b1)

    sum2 = _sc_agg_split(h2.reshape(N, 4, 16), src, dst)
    h3 = _tc_layer(sum2.reshape(N, HID), deginv, h2, Wn2, Ws2, bc2,
                   bn_g2, bn_b2)

    poolp, poolc = _sc_pool(h3, batch)
    return _tc_readout(poolp, poolc, Wh0, bh0, Wh1, bh1)


# speedup vs baseline: 1.5943x; 1.5943x over previous
import functools

import jax
import jax.numpy as jnp
from jax import lax
from jax.experimental import pallas as pl
from jax.experimental.pallas import tpu as pltpu
from jax.experimental.pallas import tpu_sc as plsc

N = 50000
E = 800000
G = 1000
HID = 64
EMB = 32
W128 = 128

NC = 2
NS = 16

_MESH = dict(core_axis_name="c", subcore_axis_name="s", num_cores=NC,
             num_subcores=NS)
_SC_PARAMS = dict(
    mesh=plsc.VectorSubcoreMesh(**_MESH),
    compiler_params=pltpu.CompilerParams(use_tc_tiling_on_sc=False),
)


def _dot_t(a, w):
    return lax.dot_general(a, w, (((1,), (1,)), ((), ())),
                           preferred_element_type=jnp.float32)


_CP = 3128
_CPLAST = N - (NS - 1) * _CP


def _per_tile_rows(s, fn):
    @pl.when(s < NS - 1)
    def _():
        fn(s * _CP, _CP)

    @pl.when(s == NS - 1)
    def _():
        fn((NS - 1) * _CP, _CPLAST)


def _copy_zero_slice(zbuf, acc, base, total, chunk):
    nfull = total // chunk
    rem = total - nfull * chunk
    for k in range(nfull):
        pltpu.sync_copy(zbuf, acc.at[pl.ds(base + k * chunk, chunk)])
    if rem:
        pltpu.sync_copy(zbuf.at[pl.ds(0, rem)],
                        acc.at[pl.ds(base + nfull * chunk, rem)])


def _zero_rows16(rows, nrows):
    z16 = jnp.zeros((16,), jnp.float32)

    def zrow(i, _):
        rows[i, :] = z16
        return 0

    lax.fori_loop(0, nrows, zrow, 0, unroll=False)


def _sc_agg16(h0p, src, dst):
    C = 800
    NCHUNK = E // C
    NW = NC * NS
    JMAX = (NCHUNK + NW - 1) // NW

    @functools.partial(
        pl.kernel,
        out_type=jax.ShapeDtypeStruct((N, W128), jnp.float32),
        scratch_types=[
            pltpu.VMEM((C,), jnp.int32),
            pltpu.VMEM((C,), jnp.int32),
            pltpu.VMEM((C, 16), jnp.float32),
            pltpu.VMEM_SHARED((N, 16), jnp.float32),
            pltpu.VMEM_SHARED((N, 16), jnp.float32),
        ],
        **_SC_PARAMS,
    )
    def k(h_hbm, src_hbm, dst_hbm, out_hbm, sidx, didx, rows, tab, acc):
        c = lax.axis_index("c")
        s = lax.axis_index("s")
        w = s * NC + c
        _zero_rows16(rows, C)
        _per_tile_rows(s, lambda base, n: (
            pltpu.sync_copy(h_hbm.at[pl.ds(base, n), pl.ds(0, 16)],
                            tab.at[pl.ds(base, n)]),
            _copy_zero_slice(rows, acc, base, n, C)))
        plsc.subcore_barrier()

        def body(j, _):
            ch = w + NW * j

            @pl.when(ch < NCHUNK)
            def _():
                off = ch * C
                pltpu.sync_copy(src_hbm.at[pl.ds(off, C)], sidx)
                pltpu.sync_copy(dst_hbm.at[pl.ds(off, C)], didx)
                pltpu.sync_copy(tab.at[sidx], rows)
                pltpu.sync_copy(rows, acc.at[didx], add=True)

            return 0

        lax.fori_loop(0, JMAX, body, 0, unroll=False)
        plsc.subcore_barrier()
        _per_tile_rows(s, lambda base, n: pltpu.sync_copy(
            acc.at[pl.ds(base, n)],
            out_hbm.at[pl.ds(base, n), pl.ds(c * 16, 16)]))

    return k(h0p, src, dst)


def _sc_agg_split(h, src, dst):
    C = 800
    NCHUNK = E // C
    JMAX = (NCHUNK + NS - 1) // NS

    @functools.partial(
        pl.kernel,
        out_type=jax.ShapeDtypeStruct((N, W128), jnp.float32),
        scratch_types=[
            pltpu.VMEM((C,), jnp.int32),
            pltpu.VMEM((C,), jnp.int32),
            pltpu.VMEM((C, 16), jnp.float32),
            pltpu.VMEM_SHARED((N, 16), jnp.float32),
            pltpu.VMEM_SHARED((N, 16), jnp.float32),
        ],
        **_SC_PARAMS,
    )
    def k(h_hbm, src_hbm, dst_hbm, out_hbm, sidx, didx, rows, tab, acc):
        c = lax.axis_index("c")
        s = lax.axis_index("s")

        for p in range(2):
            pv = c * 2 + p
            _zero_rows16(rows, C)
            _per_tile_rows(s, lambda base, n, pv=pv: (
                pltpu.sync_copy(h_hbm.at[pl.ds(base, n),
                                         pl.ds(pv * 16, 16)],
                                tab.at[pl.ds(base, n)]),
                _copy_zero_slice(rows, acc, base, n, C)))
            plsc.subcore_barrier()

            def body(j, _):
                ch = s + NS * j

                @pl.when(ch < NCHUNK)
                def _():
                    off = ch * C
                    pltpu.sync_copy(src_hbm.at[pl.ds(off, C)], sidx)
                    pltpu.sync_copy(dst_hbm.at[pl.ds(off, C)], didx)
                    pltpu.sync_copy(tab.at[sidx], rows)
                    pltpu.sync_copy(rows, acc.at[didx], add=True)

                return 0

            lax.fori_loop(0, JMAX, body, 0, unroll=False)
            plsc.subcore_barrier()
            _per_tile_rows(s, lambda base, n, pv=pv: pltpu.sync_copy(
                acc.at[pl.ds(base, n)],
                out_hbm.at[pl.ds(base, n), pl.ds(pv * 16, 16)]))
            if p == 0:
                plsc.subcore_barrier()

    return k(h, src, dst)


def _sc_pool(h3, batch):
    CH = 400
    NCHUNK = N // CH
    NW = NC * NS
    JMAX = (NCHUNK + NW - 1) // NW

    @functools.partial(
        pl.kernel,
        out_type=jax.ShapeDtypeStruct((NC, G, W128), jnp.float32),
        scratch_types=[
            pltpu.VMEM((CH,), jnp.int32),
            pltpu.VMEM((CH, HID), jnp.float32),
            pltpu.VMEM((CH, 16), jnp.float32),
            pltpu.VMEM((CH, 16), jnp.float32),
            pltpu.VMEM_SHARED((G, HID), jnp.float32),
            pltpu.VMEM_SHARED((G, 16), jnp.float32),
        ],
        **_SC_PARAMS,
    )
    def k(h_hbm, batch_hbm, out_hbm, bidx, rows, ones_v, zb16, accp, accc):
        c = lax.axis_index("c")
        s = lax.axis_index("s")
        w = s * NC + c

        def fill(i, _):
            rows[i, pl.ds(0, 16)] = jnp.zeros((16,), jnp.float32)
            rows[i, pl.ds(16, 16)] = jnp.zeros((16,), jnp.float32)
            rows[i, pl.ds(32, 16)] = jnp.zeros((16,), jnp.float32)
            rows[i, pl.ds(48, 16)] = jnp.zeros((16,), jnp.float32)
            ones_v[i, :] = jnp.ones((16,), jnp.float32)
            zb16[i, :] = jnp.zeros((16,), jnp.float32)
            return 0

        lax.fori_loop(0, CH, fill, 0, unroll=False)

        @pl.when(s == 0)
        def _():
            _copy_zero_slice(rows, accp, 0, G, CH)

        @pl.when(s == 1)
        def _():
            _copy_zero_slice(zb16, accc, 0, G, CH)

        plsc.subcore_barrier()

        def body(j, _):
            ch = w + NW * j

            @pl.when(ch < NCHUNK)
            def _():
                off = ch * CH
                pltpu.sync_copy(batch_hbm.at[pl.ds(off, CH)], bidx)
                pltpu.sync_copy(h_hbm.at[pl.ds(off, CH), pl.ds(0, HID)],
                                rows)
                pltpu.sync_copy(rows, accp.at[bidx], add=True)
                pltpu.sync_copy(ones_v, accc.at[bidx], add=True)

            return 0

        lax.fori_loop(0, JMAX, body, 0, unroll=False)
        plsc.subcore_barrier()

        @pl.when(s == 0)
        def _():
            pltpu.sync_copy(accp, out_hbm.at[c, pl.ds(0, G), pl.ds(0, HID)])

        @pl.when(s == 1)
        def _():
            pltpu.sync_copy(accc, out_hbm.at[c, pl.ds(0, G),
                                             pl.ds(HID, 16)])

    return k(h3, batch)


_BLK = 1000
_NB = N // _BLK


def _accum_stats(first, z, st_ref):
    st = jnp.concatenate([jnp.sum(z, axis=0)[None, :],
                          jnp.sum(z * z, axis=0)[None, :]], axis=0)

    @pl.when(first)
    def _():
        st_ref[...] = st

    @pl.when(jnp.logical_not(first))
    def _():
        st_ref[...] = st_ref[...] + st


def _tc_input_bn(x, g, b):
    def body(x_ref, g_ref, b_ref, o_ref, st):
        ph = pl.program_id(0)
        i = pl.program_id(1)
        xv = x_ref[...]

        @pl.when(ph == 0)
        def _():
            _accum_stats(i == 0, xv, st)

        @pl.when(ph == 1)
        def _():
            m = st[0] / N
            v = st[1] / N - m * m
            h = (xv - m) * lax.rsqrt(v + 1e-5) * g_ref[...] + b_ref[...]
            o_ref[...] = jnp.concatenate(
                [h, jnp.ones((_BLK, 1), jnp.float32),
                 jnp.zeros((_BLK, W128 - 5), jnp.float32)], axis=1)

    return pl.pallas_call(
        body, grid=(2, _NB),
        in_specs=[
            pl.BlockSpec((_BLK, 4), lambda ph, i: (i, 0)),
            pl.BlockSpec((4,), lambda ph, i: (0,)),
            pl.BlockSpec((4,), lambda ph, i: (0,)),
        ],
        out_specs=pl.BlockSpec((_BLK, W128), lambda ph, i: (i, 0)),
        out_shape=jax.ShapeDtypeStruct((N, W128), jnp.float32),
        scratch_shapes=[pltpu.VMEM((2, 4), jnp.float32)])(x, g, b)


def _tc_layer0(part0, h0p, Wn, Ws, bc, g, b):
    def body(p_ref, h_ref, wn_ref, ws_ref, bc_ref, g_ref, b_ref,
             dg_ref, o_ref, st):
        ph = pl.program_id(0)
        i = pl.program_id(1)
        pv = p_ref[...]
        psum = pv[:, 0:16] + pv[:, 16:32]
        deginv = 1.0 / jnp.maximum(psum[:, 4], 1.0)
        dg_ref[...] = deginv[None, None, :]
        agg = psum[:, :4] * deginv[:, None]
        h0 = h_ref[...][:, :4]
        z = _dot_t(agg, wn_ref[...]) + _dot_t(h0, ws_ref[...]) + bc_ref[...]

        @pl.when(ph == 0)
        def _():
            _accum_stats(i == 0, z, st)

        @pl.when(ph == 1)
        def _():
            m = st[0] / N
            v = st[1] / N - m * m
            y = (z - m) * lax.rsqrt(v + 1e-5) * g_ref[...] + b_ref[...]
            y = jnp.maximum(y, 0.0)
            o_ref[...] = jnp.concatenate(
                [y, jnp.zeros((_BLK, W128 - HID), jnp.float32)], axis=1)

    return pl.pallas_call(
        body, grid=(2, _NB),
        in_specs=[
            pl.BlockSpec((_BLK, W128), lambda ph, i: (i, 0)),
            pl.BlockSpec((_BLK, W128), lambda ph, i: (i, 0)),
            pl.BlockSpec((HID, 4), lambda ph, i: (0, 0)),
            pl.BlockSpec((HID, 4), lambda ph, i: (0, 0)),
            pl.BlockSpec((HID,), lambda ph, i: (0,)),
            pl.BlockSpec((HID,), lambda ph, i: (0,)),
            pl.BlockSpec((HID,), lambda ph, i: (0,)),
        ],
        out_specs=[
            pl.BlockSpec((1, 1, _BLK), lambda ph, i: (i, 0, 0)),
            pl.BlockSpec((_BLK, W128), lambda ph, i: (i, 0)),
        ],
        out_shape=[
            jax.ShapeDtypeStruct((_NB, 1, _BLK), jnp.float32),
            jax.ShapeDtypeStruct((N, W128), jnp.float32),
        ],
        scratch_shapes=[pltpu.VMEM((2, HID), jnp.float32)],
    )(part0, h0p, Wn, Ws, bc, g, b)


def _tc_layer(sums, deginv, h, Wn, Ws, bc, g, b):
    def body(su_ref, dg_ref, h_ref, wn_ref, ws_ref, bc_ref, g_ref, b_ref,
             o_ref, st):
        ph = pl.program_id(0)
        i = pl.program_id(1)
        agg = su_ref[...][:, :HID] * dg_ref[0, 0][:, None]
        z = (_dot_t(agg, wn_ref[...]) + _dot_t(h_ref[...][:, :HID],
                                               ws_ref[...]) + bc_ref[...])

        @pl.when(ph == 0)
        def _():
            _accum_stats(i == 0, z, st)

        @pl.when(ph == 1)
        def _():
            m = st[0] / N
            v = st[1] / N - m * m
            y = (z - m) * lax.rsqrt(v + 1e-5) * g_ref[...] + b_ref[...]
            y = jnp.maximum(y, 0.0)
            o_ref[...] = jnp.concatenate(
                [y, jnp.zeros((_BLK, W128 - HID), jnp.float32)], axis=1)

    return pl.pallas_call(
        body, grid=(2, _NB),
        in_specs=[
            pl.BlockSpec((_BLK, W128), lambda ph, i: (i, 0)),
            pl.BlockSpec((1, 1, _BLK), lambda ph, i: (i, 0, 0)),
            pl.BlockSpec((_BLK, W128), lambda ph, i: (i, 0)),
            pl.BlockSpec((HID, HID), lambda ph, i: (0, 0)),
            pl.BlockSpec((HID, HID), lambda ph, i: (0, 0)),
            pl.BlockSpec((HID,), lambda ph, i: (0,)),
            pl.BlockSpec((HID,), lambda ph, i: (0,)),
            pl.BlockSpec((HID,), lambda ph, i: (0,)),
        ],
        out_specs=pl.BlockSpec((_BLK, W128), lambda ph, i: (i, 0)),
        out_shape=jax.ShapeDtypeStruct((N, W128), jnp.float32),
        scratch_shapes=[pltpu.VMEM((2, HID), jnp.float32)],
    )(sums, deginv, h, Wn, Ws, bc, g, b)


def _tc_readout(pool, Wh0, bh0, Wh1, bh1):
    def body(p_ref, w0_ref, b0_ref, w1_ref, b1_ref, o_ref):
        pv = p_ref[0] + p_ref[1]
        pooled = pv[:, :HID]
        cnt = jnp.maximum(pv[:, HID], 1.0)
        mean = pooled / cnt[:, None]
        z = jnp.maximum(_dot_t(mean, w0_ref[...]) + b0_ref[...], 0.0)
        z2 = _dot_t(z, w1_ref[...]) + b1_ref[...]
        n = jnp.sqrt(jnp.sum(z2 * z2, axis=1, keepdims=True))
        o_ref[...] = z2 / jnp.maximum(n, 1e-12)

    return pl.pallas_call(
        body,
        out_shape=jax.ShapeDtypeStruct((G, EMB), jnp.float32))(
            pool, Wh0, bh0, Wh1, bh1)


def kernel(x, edge_index, batch, bn_in_g, bn_in_b,
           Wn0, Ws0, bc0, bn_g0, bn_b0,
           Wn1, Ws1, bc1, bn_g1, bn_b1,
           Wn2, Ws2, bc2, bn_g2, bn_b2,
           Wh0, bh0, Wh1, bh1):
    src = edge_index[0]
    dst = edge_index[1]

    h0p = _tc_input_bn(x, bn_in_g, bn_in_b)

    part0 = _sc_agg16(h0p, src, dst)
    deginv, h1 = _tc_layer0(part0, h0p, Wn0, Ws0, bc0, bn_g0, bn_b0)

    sum1 = _sc_agg_split(h1, src, dst)
    h2 = _tc_layer(sum1, deginv, h1, Wn1, Ws1, bc1, bn_g1, bn_b1)

    sum2 = _sc_agg_split(h2, src, dst)
    h3 = _tc_layer(sum2, deginv, h2, Wn2, Ws2, bc2, bn_g2, bn_b2)

    pool = _sc_pool(h3, batch)
    return _tc_readout(pool, Wh0, bh0, Wh1, bh1)


# speedup vs baseline: 1.6098x; 1.0097x over previous
import functools

import jax
import jax.numpy as jnp
from jax import lax
from jax.experimental import pallas as pl
from jax.experimental.pallas import tpu as pltpu
from jax.experimental.pallas import tpu_sc as plsc

N = 50000
E = 800000
G = 1000
HID = 64
EMB = 32
W128 = 128

NC = 2
NS = 16

_MESH = dict(core_axis_name="c", subcore_axis_name="s", num_cores=NC,
             num_subcores=NS)
_SC_PARAMS = dict(
    mesh=plsc.VectorSubcoreMesh(**_MESH),
    compiler_params=pltpu.CompilerParams(use_tc_tiling_on_sc=False),
)


def _dot_t(a, w):
    return lax.dot_general(a, w, (((1,), (1,)), ((), ())),
                           preferred_element_type=jnp.float32)


_CP = 3128
_CPLAST = N - (NS - 1) * _CP


def _per_tile_rows(s, fn):
    @pl.when(s < NS - 1)
    def _():
        fn(s * _CP, _CP)

    @pl.when(s == NS - 1)
    def _():
        fn((NS - 1) * _CP, _CPLAST)


def _copy_zero_slice(zbuf, acc, base, total, chunk):
    nfull = total // chunk
    rem = total - nfull * chunk
    for k in range(nfull):
        pltpu.sync_copy(zbuf, acc.at[pl.ds(base + k * chunk, chunk)])
    if rem:
        pltpu.sync_copy(zbuf.at[pl.ds(0, rem)],
                        acc.at[pl.ds(base + nfull * chunk, rem)])


def _zero_rows16(rows, nrows):
    z16 = jnp.zeros((16,), jnp.float32)

    def zrow(i, _):
        rows[i, :] = z16
        return 0

    lax.fori_loop(0, nrows, zrow, 0, unroll=False)


def _sc_agg16(h0p, src, dst):
    C = 800
    NCHUNK = E // C
    NW = NC * NS
    JMAX = (NCHUNK + NW - 1) // NW

    @functools.partial(
        pl.kernel,
        out_type=jax.ShapeDtypeStruct((N, W128), jnp.float32),
        scratch_types=[
            pltpu.VMEM((C,), jnp.int32),
            pltpu.VMEM((C,), jnp.int32),
            pltpu.VMEM((C, 16), jnp.float32),
            pltpu.VMEM_SHARED((N, 16), jnp.float32),
            pltpu.VMEM_SHARED((N, 16), jnp.float32),
        ],
        **_SC_PARAMS,
    )
    def k(h_hbm, src_hbm, dst_hbm, out_hbm, sidx, didx, rows, tab, acc):
        c = lax.axis_index("c")
        s = lax.axis_index("s")
        w = s * NC + c
        _zero_rows16(rows, C)
        _per_tile_rows(s, lambda base, n: (
            pltpu.sync_copy(h_hbm.at[pl.ds(base, n), pl.ds(0, 16)],
                            tab.at[pl.ds(base, n)]),
            _copy_zero_slice(rows, acc, base, n, C)))
        plsc.subcore_barrier()

        def body(j, _):
            ch = w + NW * j

            @pl.when(ch < NCHUNK)
            def _():
                off = ch * C
                pltpu.sync_copy(src_hbm.at[pl.ds(off, C)], sidx)
                pltpu.sync_copy(dst_hbm.at[pl.ds(off, C)], didx)
                pltpu.sync_copy(tab.at[sidx], rows)
                pltpu.sync_copy(rows, acc.at[didx], add=True)

            return 0

        lax.fori_loop(0, JMAX, body, 0, unroll=False)
        plsc.subcore_barrier()
        _per_tile_rows(s, lambda base, n: pltpu.sync_copy(
            acc.at[pl.ds(base, n)],
            out_hbm.at[pl.ds(base, n), pl.ds(c * 16, 16)]))

    return k(h0p, src, dst)


def _sc_agg_split(h, src, dst):
    C = 400
    EPT = E // NS
    NIT = EPT // C
    NPAIR = NIT // 2

    @functools.partial(
        pl.kernel,
        out_type=jax.ShapeDtypeStruct((N, W128), jnp.float32),
        scratch_types=[
            pltpu.VMEM((C,), jnp.int32),
            pltpu.VMEM((C,), jnp.int32),
            pltpu.VMEM((C,), jnp.int32),
            pltpu.VMEM((C,), jnp.int32),
            pltpu.VMEM((C, 16), jnp.float32),
            pltpu.VMEM((C, 16), jnp.float32),
            pltpu.VMEM_SHARED((N, 16), jnp.float32),
            pltpu.VMEM_SHARED((N, 16), jnp.float32),
            pltpu.SemaphoreType.DMA,
            pltpu.SemaphoreType.DMA,
        ],
        **_SC_PARAMS,
    )
    def k(h_hbm, src_hbm, dst_hbm, out_hbm,
          s0, d0, s1, d1, r0, r1, tab, acc, sem_i, sem_s):
        c = lax.axis_index("c")
        s = lax.axis_index("s")
        base_e = s * EPT

        for p in range(2):
            pv = c * 2 + p
            _zero_rows16(r0, C)
            _per_tile_rows(s, lambda base, n, pv=pv: (
                pltpu.sync_copy(h_hbm.at[pl.ds(base, n),
                                         pl.ds(pv * 16, 16)],
                                tab.at[pl.ds(base, n)]),
                _copy_zero_slice(r0, acc, base, n, C)))
            plsc.subcore_barrier()

            def pair(jj, _):
                offa = base_e + jj * (2 * C)
                offb = offa + C
                pltpu.sync_copy(src_hbm.at[pl.ds(offa, C)], s0)
                pltpu.sync_copy(dst_hbm.at[pl.ds(offa, C)], d0)
                ib0 = pltpu.async_copy(src_hbm.at[pl.ds(offb, C)], s1, sem_i)
                ib1 = pltpu.async_copy(dst_hbm.at[pl.ds(offb, C)], d1, sem_i)
                pltpu.sync_copy(tab.at[s0], r0)
                sa = pltpu.async_copy(r0, acc.at[d0], sem_s, add=True)
                ib0.wait()
                ib1.wait()
                pltpu.sync_copy(tab.at[s1], r1)
                sa.wait()
                sb = pltpu.async_copy(r1, acc.at[d1], sem_s, add=True)
                sb.wait()
                return 0

            lax.fori_loop(0, NPAIR, pair, 0, unroll=False)
            offt = base_e + (NIT - 1) * C
            pltpu.sync_copy(src_hbm.at[pl.ds(offt, C)], s0)
            pltpu.sync_copy(dst_hbm.at[pl.ds(offt, C)], d0)
            pltpu.sync_copy(tab.at[s0], r0)
            pltpu.sync_copy(r0, acc.at[d0], add=True)

            plsc.subcore_barrier()
            _per_tile_rows(s, lambda base, n, pv=pv: pltpu.sync_copy(
                acc.at[pl.ds(base, n)],
                out_hbm.at[pl.ds(base, n), pl.ds(pv * 16, 16)]))
            if p == 0:
                plsc.subcore_barrier()

    return k(h, src, dst)


def _sc_pool(h3, batch):
    CH = 400
    NCHUNK = N // CH
    NW = NC * NS
    JMAX = (NCHUNK + NW - 1) // NW

    @functools.partial(
        pl.kernel,
        out_type=jax.ShapeDtypeStruct((NC, G, W128), jnp.float32),
        scratch_types=[
            pltpu.VMEM((CH,), jnp.int32),
            pltpu.VMEM((CH, HID), jnp.float32),
            pltpu.VMEM((CH, 16), jnp.float32),
            pltpu.VMEM((CH, 16), jnp.float32),
            pltpu.VMEM_SHARED((G, HID), jnp.float32),
            pltpu.VMEM_SHARED((G, 16), jnp.float32),
        ],
        **_SC_PARAMS,
    )
    def k(h_hbm, batch_hbm, out_hbm, bidx, rows, ones_v, zb16, accp, accc):
        c = lax.axis_index("c")
        s = lax.axis_index("s")
        w = s * NC + c

        def fill(i, _):
            rows[i, pl.ds(0, 16)] = jnp.zeros((16,), jnp.float32)
            rows[i, pl.ds(16, 16)] = jnp.zeros((16,), jnp.float32)
            rows[i, pl.ds(32, 16)] = jnp.zeros((16,), jnp.float32)
            rows[i, pl.ds(48, 16)] = jnp.zeros((16,), jnp.float32)
            ones_v[i, :] = jnp.ones((16,), jnp.float32)
            zb16[i, :] = jnp.zeros((16,), jnp.float32)
            return 0

        lax.fori_loop(0, CH, fill, 0, unroll=False)

        @pl.when(s == 0)
        def _():
            _copy_zero_slice(rows, accp, 0, G, CH)

        @pl.when(s == 1)
        def _():
            _copy_zero_slice(zb16, accc, 0, G, CH)

        plsc.subcore_barrier()

        def body(j, _):
            ch = w + NW * j

            @pl.when(ch < NCHUNK)
            def _():
                off = ch * CH
                pltpu.sync_copy(batch_hbm.at[pl.ds(off, CH)], bidx)
                pltpu.sync_copy(h_hbm.at[pl.ds(off, CH), pl.ds(0, HID)],
                                rows)
                pltpu.sync_copy(rows, accp.at[bidx], add=True)
                pltpu.sync_copy(ones_v, accc.at[bidx], add=True)

            return 0

        lax.fori_loop(0, JMAX, body, 0, unroll=False)
        plsc.subcore_barrier()

        @pl.when(s == 0)
        def _():
            pltpu.sync_copy(accp, out_hbm.at[c, pl.ds(0, G), pl.ds(0, HID)])

        @pl.when(s == 1)
        def _():
            pltpu.sync_copy(accc, out_hbm.at[c, pl.ds(0, G),
                                             pl.ds(HID, 16)])

    return k(h3, batch)


_BLK = 1000
_NB = N // _BLK


def _accum_stats(first, z, st_ref):
    st = jnp.concatenate([jnp.sum(z, axis=0)[None, :],
                          jnp.sum(z * z, axis=0)[None, :]], axis=0)

    @pl.when(first)
    def _():
        st_ref[...] = st

    @pl.when(jnp.logical_not(first))
    def _():
        st_ref[...] = st_ref[...] + st


def _tc_input_bn(x, g, b):
    def body(x_ref, g_ref, b_ref, o_ref, st):
        ph = pl.program_id(0)
        i = pl.program_id(1)
        xv = x_ref[...]

        @pl.when(ph == 0)
        def _():
            _accum_stats(i == 0, xv, st)

        @pl.when(ph == 1)
        def _():
            m = st[0] / N
            v = st[1] / N - m * m
            h = (xv - m) * lax.rsqrt(v + 1e-5) * g_ref[...] + b_ref[...]
            o_ref[...] = jnp.concatenate(
                [h, jnp.ones((_BLK, 1), jnp.float32),
                 jnp.zeros((_BLK, W128 - 5), jnp.float32)], axis=1)

    return pl.pallas_call(
        body, grid=(2, _NB),
        in_specs=[
            pl.BlockSpec((_BLK, 4), lambda ph, i: (i, 0)),
            pl.BlockSpec((4,), lambda ph, i: (0,)),
            pl.BlockSpec((4,), lambda ph, i: (0,)),
        ],
        out_specs=pl.BlockSpec((_BLK, W128), lambda ph, i: (i, 0)),
        out_shape=jax.ShapeDtypeStruct((N, W128), jnp.float32),
        scratch_shapes=[pltpu.VMEM((2, 4), jnp.float32)])(x, g, b)


def _tc_layer0(part0, h0p, Wn, Ws, bc, g, b):
    def body(p_ref, h_ref, wn_ref, ws_ref, bc_ref, g_ref, b_ref,
             dg_ref, o_ref, st):
        ph = pl.program_id(0)
        i = pl.program_id(1)
        pv = p_ref[...]
        psum = pv[:, 0:16] + pv[:, 16:32]
        deginv = 1.0 / jnp.maximum(psum[:, 4], 1.0)
        dg_ref[...] = deginv[None, None, :]
        agg = psum[:, :4] * deginv[:, None]
        h0 = h_ref[...][:, :4]
        z = _dot_t(agg, wn_ref[...]) + _dot_t(h0, ws_ref[...]) + bc_ref[...]

        @pl.when(ph == 0)
        def _():
            _accum_stats(i == 0, z, st)

        @pl.when(ph == 1)
        def _():
            m = st[0] / N
            v = st[1] / N - m * m
            y = (z - m) * lax.rsqrt(v + 1e-5) * g_ref[...] + b_ref[...]
            y = jnp.maximum(y, 0.0)
            o_ref[...] = jnp.concatenate(
                [y, jnp.zeros((_BLK, W128 - HID), jnp.float32)], axis=1)

    return pl.pallas_call(
        body, grid=(2, _NB),
        in_specs=[
            pl.BlockSpec((_BLK, W128), lambda ph, i: (i, 0)),
            pl.BlockSpec((_BLK, W128), lambda ph, i: (i, 0)),
            pl.BlockSpec((HID, 4), lambda ph, i: (0, 0)),
            pl.BlockSpec((HID, 4), lambda ph, i: (0, 0)),
            pl.BlockSpec((HID,), lambda ph, i: (0,)),
            pl.BlockSpec((HID,), lambda ph, i: (0,)),
            pl.BlockSpec((HID,), lambda ph, i: (0,)),
        ],
        out_specs=[
            pl.BlockSpec((1, 1, _BLK), lambda ph, i: (i, 0, 0)),
            pl.BlockSpec((_BLK, W128), lambda ph, i: (i, 0)),
        ],
        out_shape=[
            jax.ShapeDtypeStruct((_NB, 1, _BLK), jnp.float32),
            jax.ShapeDtypeStruct((N, W128), jnp.float32),
        ],
        scratch_shapes=[pltpu.VMEM((2, HID), jnp.float32)],
    )(part0, h0p, Wn, Ws, bc, g, b)


def _tc_layer(sums, deginv, h, Wn, Ws, bc, g, b):
    def body(su_ref, dg_ref, h_ref, wn_ref, ws_ref, bc_ref, g_ref, b_ref,
             o_ref, st):
        ph = pl.program_id(0)
        i = pl.program_id(1)
        agg = su_ref[...][:, :HID] * dg_ref[0, 0][:, None]
        z = (_dot_t(agg, wn_ref[...]) + _dot_t(h_ref[...][:, :HID],
                                               ws_ref[...]) + bc_ref[...])

        @pl.when(ph == 0)
        def _():
            _accum_stats(i == 0, z, st)

        @pl.when(ph == 1)
        def _():
            m = st[0] / N
            v = st[1] / N - m * m
            y = (z - m) * lax.rsqrt(v + 1e-5) * g_ref[...] + b_ref[...]
            y = jnp.maximum(y, 0.0)
            o_ref[...] = jnp.concatenate(
                [y, jnp.zeros((_BLK, W128 - HID), jnp.float32)], axis=1)

    return pl.pallas_call(
        body, grid=(2, _NB),
        in_specs=[
            pl.BlockSpec((_BLK, W128), lambda ph, i: (i, 0)),
            pl.BlockSpec((1, 1, _BLK), lambda ph, i: (i, 0, 0)),
            pl.BlockSpec((_BLK, W128), lambda ph, i: (i, 0)),
            pl.BlockSpec((HID, HID), lambda ph, i: (0, 0)),
            pl.BlockSpec((HID, HID), lambda ph, i: (0, 0)),
            pl.BlockSpec((HID,), lambda ph, i: (0,)),
            pl.BlockSpec((HID,), lambda ph, i: (0,)),
            pl.BlockSpec((HID,), lambda ph, i: (0,)),
        ],
        out_specs=pl.BlockSpec((_BLK, W128), lambda ph, i: (i, 0)),
        out_shape=jax.ShapeDtypeStruct((N, W128), jnp.float32),
        scratch_shapes=[pltpu.VMEM((2, HID), jnp.float32)],
    )(sums, deginv, h, Wn, Ws, bc, g, b)


def _tc_readout(pool, Wh0, bh0, Wh1, bh1):
    def body(p_ref, w0_ref, b0_ref, w1_ref, b1_ref, o_ref):
        pv = p_ref[0] + p_ref[1]
        pooled = pv[:, :HID]
        cnt = jnp.maximum(pv[:, HID], 1.0)
        mean = pooled / cnt[:, None]
        z = jnp.maximum(_dot_t(mean, w0_ref[...]) + b0_ref[...], 0.0)
        z2 = _dot_t(z, w1_ref[...]) + b1_ref[...]
        n = jnp.sqrt(jnp.sum(z2 * z2, axis=1, keepdims=True))
        o_ref[...] = z2 / jnp.maximum(n, 1e-12)

    return pl.pallas_call(
        body,
        out_shape=jax.ShapeDtypeStruct((G, EMB), jnp.float32))(
            pool, Wh0, bh0, Wh1, bh1)


def kernel(x, edge_index, batch, bn_in_g, bn_in_b,
           Wn0, Ws0, bc0, bn_g0, bn_b0,
           Wn1, Ws1, bc1, bn_g1, bn_b1,
           Wn2, Ws2, bc2, bn_g2, bn_b2,
           Wh0, bh0, Wh1, bh1):
    src = edge_index[0]
    dst = edge_index[1]

    h0p = _tc_input_bn(x, bn_in_g, bn_in_b)

    part0 = _sc_agg16(h0p, src, dst)
    deginv, h1 = _tc_layer0(part0, h0p, Wn0, Ws0, bc0, bn_g0, bn_b0)

    sum1 = _sc_agg_split(h1, src, dst)
    h2 = _tc_layer(sum1, deginv, h1, Wn1, Ws1, bc1, bn_g1, bn_b1)

    sum2 = _sc_agg_split(h2, src, dst)
    h3 = _tc_layer(sum2, deginv, h2, Wn2, Ws2, bc2, bn_g2, bn_b2)

    pool = _sc_pool(h3, batch)
    return _tc_readout(pool, Wh0, bh0, Wh1, bh1)


# speedup vs baseline: 1.7489x; 1.0864x over previous
import functools

import jax
import jax.numpy as jnp
from jax import lax
from jax.experimental import pallas as pl
from jax.experimental.pallas import tpu as pltpu
from jax.experimental.pallas import tpu_sc as plsc

N = 50000
E = 800000
G = 1000
HID = 64
EMB = 32
W128 = 128

NC = 2
NS = 16

_MESH = dict(core_axis_name="c", subcore_axis_name="s", num_cores=NC,
             num_subcores=NS)
_SC_PARAMS = dict(
    mesh=plsc.VectorSubcoreMesh(**_MESH),
    compiler_params=pltpu.CompilerParams(use_tc_tiling_on_sc=False),
)


def _dot_t(a, w):
    return lax.dot_general(a, w, (((1,), (1,)), ((), ())),
                           preferred_element_type=jnp.float32)


_CP = 3128
_CPLAST = N - (NS - 1) * _CP


def _per_tile_rows(s, fn):
    @pl.when(s < NS - 1)
    def _():
        fn(s * _CP, _CP)

    @pl.when(s == NS - 1)
    def _():
        fn((NS - 1) * _CP, _CPLAST)


def _copy_zero_slice(zbuf, acc, base, total, chunk):
    nfull = total // chunk
    rem = total - nfull * chunk
    for k in range(nfull):
        pltpu.sync_copy(zbuf, acc.at[pl.ds(base + k * chunk, chunk)])
    if rem:
        pltpu.sync_copy(zbuf.at[pl.ds(0, rem)],
                        acc.at[pl.ds(base + nfull * chunk, rem)])


def _zero_rows16(rows, nrows):
    z16 = jnp.zeros((16,), jnp.float32)

    def zrow(i, _):
        rows[i, :] = z16
        return 0

    lax.fori_loop(0, nrows, zrow, 0, unroll=False)


def _sc_agg16(h0p, src, dst):
    C = 800
    NCHUNK = E // C
    NW = NC * NS
    JMAX = (NCHUNK + NW - 1) // NW

    @functools.partial(
        pl.kernel,
        out_type=jax.ShapeDtypeStruct((N, W128), jnp.float32),
        scratch_types=[
            pltpu.VMEM((C,), jnp.int32),
            pltpu.VMEM((C,), jnp.int32),
            pltpu.VMEM((C, 16), jnp.float32),
            pltpu.VMEM_SHARED((N, 16), jnp.float32),
            pltpu.VMEM_SHARED((N, 16), jnp.float32),
        ],
        **_SC_PARAMS,
    )
    def k(h_hbm, src_hbm, dst_hbm, out_hbm, sidx, didx, rows, tab, acc):
        c = lax.axis_index("c")
        s = lax.axis_index("s")
        w = s * NC + c
        _zero_rows16(rows, C)
        _per_tile_rows(s, lambda base, n: (
            pltpu.sync_copy(h_hbm.at[pl.ds(base, n), pl.ds(0, 16)],
                            tab.at[pl.ds(base, n)]),
            _copy_zero_slice(rows, acc, base, n, C)))
        plsc.subcore_barrier()

        def body(j, _):
            ch = w + NW * j

            @pl.when(ch < NCHUNK)
            def _():
                off = ch * C
                pltpu.sync_copy(src_hbm.at[pl.ds(off, C)], sidx)
                pltpu.sync_copy(dst_hbm.at[pl.ds(off, C)], didx)
                pltpu.sync_copy(tab.at[sidx], rows)
                pltpu.sync_copy(rows, acc.at[didx], add=True)

            return 0

        lax.fori_loop(0, JMAX, body, 0, unroll=False)
        plsc.subcore_barrier()
        _per_tile_rows(s, lambda base, n: pltpu.sync_copy(
            acc.at[pl.ds(base, n)],
            out_hbm.at[pl.ds(base, n), pl.ds(c * 16, 16)]))

    return k(h0p, src, dst)


def _sc_agg_split(h, src, dst):
    C = 400
    EPT = E // NS
    NIT = EPT // C
    NPAIR = NIT // 2

    @functools.partial(
        pl.kernel,
        out_type=jax.ShapeDtypeStruct((N, W128), jnp.float32),
        scratch_types=[
            pltpu.VMEM((C,), jnp.int32),
            pltpu.VMEM((C,), jnp.int32),
            pltpu.VMEM((C,), jnp.int32),
            pltpu.VMEM((C,), jnp.int32),
            pltpu.VMEM((C, 16), jnp.float32),
            pltpu.VMEM((C, 16), jnp.float32),
            pltpu.VMEM_SHARED((N, 16), jnp.float32),
            pltpu.VMEM_SHARED((N, 16), jnp.float32),
            pltpu.SemaphoreType.DMA,
            pltpu.SemaphoreType.DMA,
        ],
        **_SC_PARAMS,
    )
    def k(h_hbm, src_hbm, dst_hbm, out_hbm,
          s0, d0, s1, d1, r0, r1, tab, acc, sem_i, sem_s):
        c = lax.axis_index("c")
        s = lax.axis_index("s")
        base_e = s * EPT

        for p in range(2):
            pv = c * 2 + p
            _zero_rows16(r0, C)
            _per_tile_rows(s, lambda base, n, pv=pv: (
                pltpu.sync_copy(h_hbm.at[pl.ds(base, n),
                                         pl.ds(pv * 16, 16)],
                                tab.at[pl.ds(base, n)]),
                _copy_zero_slice(r0, acc, base, n, C)))
            plsc.subcore_barrier()

            def pair(jj, _):
                offa = base_e + jj * (2 * C)
                offb = offa + C
                pltpu.sync_copy(src_hbm.at[pl.ds(offa, C)], s0)
                pltpu.sync_copy(dst_hbm.at[pl.ds(offa, C)], d0)
                ib0 = pltpu.async_copy(src_hbm.at[pl.ds(offb, C)], s1, sem_i)
                ib1 = pltpu.async_copy(dst_hbm.at[pl.ds(offb, C)], d1, sem_i)
                pltpu.sync_copy(tab.at[s0], r0)
                sa = pltpu.async_copy(r0, acc.at[d0], sem_s, add=True)
                ib0.wait()
                ib1.wait()
                pltpu.sync_copy(tab.at[s1], r1)
                sa.wait()
                sb = pltpu.async_copy(r1, acc.at[d1], sem_s, add=True)
                sb.wait()
                return 0

            lax.fori_loop(0, NPAIR, pair, 0, unroll=False)
            offt = base_e + (NIT - 1) * C
            pltpu.sync_copy(src_hbm.at[pl.ds(offt, C)], s0)
            pltpu.sync_copy(dst_hbm.at[pl.ds(offt, C)], d0)
            pltpu.sync_copy(tab.at[s0], r0)
            pltpu.sync_copy(r0, acc.at[d0], add=True)

            plsc.subcore_barrier()
            _per_tile_rows(s, lambda base, n, pv=pv: pltpu.sync_copy(
                acc.at[pl.ds(base, n)],
                out_hbm.at[pl.ds(base, n), pl.ds(pv * 16, 16)]))
            if p == 0:
                plsc.subcore_barrier()

    return k(h, src, dst)


def _sc_pool(h3, batch):
    CH = 400
    NCHUNK = N // CH
    NW = NC * NS
    JMAX = (NCHUNK + NW - 1) // NW

    @functools.partial(
        pl.kernel,
        out_type=jax.ShapeDtypeStruct((NC, G, W128), jnp.float32),
        scratch_types=[
            pltpu.VMEM((CH,), jnp.int32),
            pltpu.VMEM((CH, HID), jnp.float32),
            pltpu.VMEM((CH, 16), jnp.float32),
            pltpu.VMEM((CH, 16), jnp.float32),
            pltpu.VMEM_SHARED((G, HID), jnp.float32),
            pltpu.VMEM_SHARED((G, 16), jnp.float32),
        ],
        **_SC_PARAMS,
    )
    def k(h_hbm, batch_hbm, out_hbm, bidx, rows, ones_v, zb16, accp, accc):
        c = lax.axis_index("c")
        s = lax.axis_index("s")
        w = s * NC + c

        def fill(i, _):
            rows[i, pl.ds(0, 16)] = jnp.zeros((16,), jnp.float32)
            rows[i, pl.ds(16, 16)] = jnp.zeros((16,), jnp.float32)
            rows[i, pl.ds(32, 16)] = jnp.zeros((16,), jnp.float32)
            rows[i, pl.ds(48, 16)] = jnp.zeros((16,), jnp.float32)
            ones_v[i, :] = jnp.ones((16,), jnp.float32)
            zb16[i, :] = jnp.zeros((16,), jnp.float32)
            return 0

        lax.fori_loop(0, CH, fill, 0, unroll=False)

        @pl.when(s == 0)
        def _():
            _copy_zero_slice(rows, accp, 0, G, CH)

        @pl.when(s == 1)
        def _():
            _copy_zero_slice(zb16, accc, 0, G, CH)

        plsc.subcore_barrier()

        def body(j, _):
            ch = w + NW * j

            @pl.when(ch < NCHUNK)
            def _():
                off = ch * CH
                pltpu.sync_copy(batch_hbm.at[pl.ds(off, CH)], bidx)
                pltpu.sync_copy(h_hbm.at[pl.ds(off, CH), pl.ds(0, HID)],
                                rows)
                pltpu.sync_copy(rows, accp.at[bidx], add=True)
                pltpu.sync_copy(ones_v, accc.at[bidx], add=True)

            return 0

        lax.fori_loop(0, JMAX, body, 0, unroll=False)
        plsc.subcore_barrier()

        @pl.when(s == 0)
        def _():
            pltpu.sync_copy(accp, out_hbm.at[c, pl.ds(0, G), pl.ds(0, HID)])

        @pl.when(s == 1)
        def _():
            pltpu.sync_copy(accc, out_hbm.at[c, pl.ds(0, G),
                                             pl.ds(HID, 16)])

    return k(h3, batch)


_BLK = 1000
_NB = N // _BLK


def _accum_stats(first, z, st_ref):
    st = jnp.concatenate([jnp.sum(z, axis=0)[None, :],
                          jnp.sum(z * z, axis=0)[None, :]], axis=0)

    @pl.when(first)
    def _():
        st_ref[...] = st

    @pl.when(jnp.logical_not(first))
    def _():
        st_ref[...] = st_ref[...] + st


def _tc_input_bn(x, g, b):
    def body(x_ref, g_ref, b_ref, o_ref, st, xbuf):
        ph = pl.program_id(0)
        i = pl.program_id(1)

        @pl.when(ph == 0)
        def _():
            xv = x_ref[...]
            xbuf[pl.ds(i * _BLK, _BLK), :] = xv
            _accum_stats(i == 0, xv, st)

        @pl.when(ph == 1)
        def _():
            xv = xbuf[pl.ds(i * _BLK, _BLK), :]
            m = st[0] / N
            v = st[1] / N - m * m
            h = (xv - m) * lax.rsqrt(v + 1e-5) * g_ref[...] + b_ref[...]
            o_ref[...] = jnp.concatenate(
                [h, jnp.ones((_BLK, 1), jnp.float32),
                 jnp.zeros((_BLK, W128 - 5), jnp.float32)], axis=1)

    return pl.pallas_call(
        body, grid=(2, _NB),
        in_specs=[
            pl.BlockSpec((_BLK, 4), lambda ph, i: (i * (1 - ph), 0)),
            pl.BlockSpec((4,), lambda ph, i: (0,)),
            pl.BlockSpec((4,), lambda ph, i: (0,)),
        ],
        out_specs=pl.BlockSpec((_BLK, W128), lambda ph, i: (i * ph, 0)),
        out_shape=jax.ShapeDtypeStruct((N, W128), jnp.float32),
        scratch_shapes=[pltpu.VMEM((2, 4), jnp.float32),
                        pltpu.VMEM((N, 4), jnp.float32)])(x, g, b)


def _tc_layer0(part0, h0p, Wn, Ws, bc, g, b):
    def body(p_ref, h_ref, wn_ref, ws_ref, bc_ref, g_ref, b_ref,
             dg_ref, o_ref, st, zbuf):
        ph = pl.program_id(0)
        i = pl.program_id(1)
        pv = p_ref[...]
        psum = pv[:, 0:16] + pv[:, 16:32]
        deginv = 1.0 / jnp.maximum(psum[:, 4], 1.0)
        dg_ref[...] = deginv[None, None, :]

        @pl.when(ph == 0)
        def _():
            agg = psum[:, :4] * deginv[:, None]
            h0 = h_ref[...][:, :4]
            z = (_dot_t(agg, wn_ref[...]) + _dot_t(h0, ws_ref[...])
                 + bc_ref[...])
            zbuf[pl.ds(i * _BLK, _BLK), :] = z
            _accum_stats(i == 0, z, st)

        @pl.when(ph == 1)
        def _():
            z = zbuf[pl.ds(i * _BLK, _BLK), :]
            m = st[0] / N
            v = st[1] / N - m * m
            y = (z - m) * lax.rsqrt(v + 1e-5) * g_ref[...] + b_ref[...]
            y = jnp.maximum(y, 0.0)
            o_ref[...] = jnp.concatenate(
                [y, jnp.zeros((_BLK, W128 - HID), jnp.float32)], axis=1)

    return pl.pallas_call(
        body, grid=(2, _NB),
        in_specs=[
            pl.BlockSpec((_BLK, W128), lambda ph, i: (i, 0)),
            pl.BlockSpec((_BLK, W128), lambda ph, i: (i * (1 - ph), 0)),
            pl.BlockSpec((HID, 4), lambda ph, i: (0, 0)),
            pl.BlockSpec((HID, 4), lambda ph, i: (0, 0)),
            pl.BlockSpec((HID,), lambda ph, i: (0,)),
            pl.BlockSpec((HID,), lambda ph, i: (0,)),
            pl.BlockSpec((HID,), lambda ph, i: (0,)),
        ],
        out_specs=[
            pl.BlockSpec((1, 1, _BLK), lambda ph, i: (i, 0, 0)),
            pl.BlockSpec((_BLK, W128), lambda ph, i: (i * ph, 0)),
        ],
        out_shape=[
            jax.ShapeDtypeStruct((_NB, 1, _BLK), jnp.float32),
            jax.ShapeDtypeStruct((N, W128), jnp.float32),
        ],
        scratch_shapes=[pltpu.VMEM((2, HID), jnp.float32),
                        pltpu.VMEM((N, HID), jnp.float32)],
    )(part0, h0p, Wn, Ws, bc, g, b)


def _tc_layer(sums, deginv, h, Wn, Ws, bc, g, b):
    def body(su_ref, dg_ref, h_ref, wn_ref, ws_ref, bc_ref, g_ref, b_ref,
             o_ref, st, zbuf):
        ph = pl.program_id(0)
        i = pl.program_id(1)

        @pl.when(ph == 0)
        def _():
            agg = su_ref[...][:, :HID] * dg_ref[0, 0][:, None]
            z = (_dot_t(agg, wn_ref[...]) + _dot_t(h_ref[...][:, :HID],
                                                   ws_ref[...]) + bc_ref[...])
            zbuf[pl.ds(i * _BLK, _BLK), :] = z
            _accum_stats(i == 0, z, st)

        @pl.when(ph == 1)
        def _():
            z = zbuf[pl.ds(i * _BLK, _BLK), :]
            m = st[0] / N
            v = st[1] / N - m * m
            y = (z - m) * lax.rsqrt(v + 1e-5) * g_ref[...] + b_ref[...]
            y = jnp.maximum(y, 0.0)
            o_ref[...] = jnp.concatenate(
                [y, jnp.zeros((_BLK, W128 - HID), jnp.float32)], axis=1)

    return pl.pallas_call(
        body, grid=(2, _NB),
        in_specs=[
            pl.BlockSpec((_BLK, W128), lambda ph, i: (i * (1 - ph), 0)),
            pl.BlockSpec((1, 1, _BLK), lambda ph, i: (i * (1 - ph), 0, 0)),
            pl.BlockSpec((_BLK, W128), lambda ph, i: (i * (1 - ph), 0)),
            pl.BlockSpec((HID, HID), lambda ph, i: (0, 0)),
            pl.BlockSpec((HID, HID), lambda ph, i: (0, 0)),
            pl.BlockSpec((HID,), lambda ph, i: (0,)),
            pl.BlockSpec((HID,), lambda ph, i: (0,)),
            pl.BlockSpec((HID,), lambda ph, i: (0,)),
        ],
        out_specs=pl.BlockSpec((_BLK, W128), lambda ph, i: (i * ph, 0)),
        out_shape=jax.ShapeDtypeStruct((N, W128), jnp.float32),
        scratch_shapes=[pltpu.VMEM((2, HID), jnp.float32),
                        pltpu.VMEM((N, HID), jnp.float32)],
    )(sums, deginv, h, Wn, Ws, bc, g, b)


def _tc_readout(pool, Wh0, bh0, Wh1, bh1):
    def body(p_ref, w0_ref, b0_ref, w1_ref, b1_ref, o_ref):
        pv = p_ref[0] + p_ref[1]
        pooled = pv[:, :HID]
        cnt = jnp.maximum(pv[:, HID], 1.0)
        mean = pooled / cnt[:, None]
        z = jnp.maximum(_dot_t(mean, w0_ref[...]) + b0_ref[...], 0.0)
        z2 = _dot_t(z, w1_ref[...]) + b1_ref[...]
        n = jnp.sqrt(jnp.sum(z2 * z2, axis=1, keepdims=True))
        o_ref[...] = z2 / jnp.maximum(n, 1e-12)

    return pl.pallas_call(
        body,
        out_shape=jax.ShapeDtypeStruct((G, EMB), jnp.float32))(
            pool, Wh0, bh0, Wh1, bh1)


def kernel(x, edge_index, batch, bn_in_g, bn_in_b,
           Wn0, Ws0, bc0, bn_g0, bn_b0,
           Wn1, Ws1, bc1, bn_g1, bn_b1,
           Wn2, Ws2, bc2, bn_g2, bn_b2,
           Wh0, bh0, Wh1, bh1):
    src = edge_index[0]
    dst = edge_index[1]

    h0p = _tc_input_bn(x, bn_in_g, bn_in_b)

    part0 = _sc_agg16(h0p, src, dst)
    deginv, h1 = _tc_layer0(part0, h0p, Wn0, Ws0, bc0, bn_g0, bn_b0)

    sum1 = _sc_agg_split(h1, src, dst)
    h2 = _tc_layer(sum1, deginv, h1, Wn1, Ws1, bc1, bn_g1, bn_b1)

    sum2 = _sc_agg_split(h2, src, dst)
    h3 = _tc_layer(sum2, deginv, h2, Wn2, Ws2, bc2, bn_g2, bn_b2)

    pool = _sc_pool(h3, batch)
    return _tc_readout(pool, Wh0, bh0, Wh1, bh1)
